# baseline (speedup 1.0000x reference)
import functools

import jax
import jax.numpy as jnp
from jax import lax
from jax.experimental import pallas as pl
from jax.experimental.pallas import tpu as pltpu
from jax.experimental.pallas import tpu_sc as plsc

N = 10000
E = 320000
D_IN = 128
D_EDGE = 16
D_OUT = 128

NC = 2
NS = 16
NW = NC * NS
LANES = 16

EW = E // NW
CH = 80
NCHUNK = EW // CH
ROWS_MAIN = 640
ROWS_LAST = N - (NS - 1) * ROWS_MAIN
ZR = 16
BN = 1000
BE = 8000


def _proj_body(nf_ref, wn_ref, bn_ref, ef_in_ref, we_ref, be_ref,
               h_ref, ef_ref):
    @pl.when(pl.program_id(0) < N // BN)
    def _():
        h_ref[...] = (
            lax.dot_general(
                nf_ref[...], wn_ref[...],
                (((1,), (1,)), ((), ())),
                preferred_element_type=jnp.float32,
            )
            + bn_ref[...]
        )

    ef_ref[...] = (
        lax.dot_general(
            ef_in_ref[...].astype(jnp.bfloat16),
            we_ref[...].astype(jnp.bfloat16),
            (((1,), (1,)), ((), ())),
            preferred_element_type=jnp.float32,
        )
        + be_ref[...]
    )


def _combine_body(p_ref, h_ref, resw_ref, degs_ref, o_ref):
    h = h_ref[...]
    res = jnp.maximum(h + resw_ref[...], 0.0) * (1.0 / degs_ref[...])
    o_ref[...] = p_ref[0] + p_ref[1] + res


def _sc_edge_body(meta_hbm, ef_hbm, h_hbm, out_hbm,
                  metab0, metab1, rows0, rows1, ef0, ef1, zbuf, agg,
                  sem_m0, sem_m1, sem_g, sem_e):
    cid = lax.axis_index("c")
    sid = lax.axis_index("s")
    wid = cid * NS + sid

    metab = (metab0, metab1)
    rows = (rows0, rows1)
    efb = (ef0, ef1)
    sem_m = (sem_m0, sem_m1)

    ebase = wid * EW

    def _meta_load(ci, b):
        off = (wid * NCHUNK + ci) * (3 * CH)
        pltpu.async_copy(
            meta_hbm.at[pl.ds(off, 3 * CH)],
            metab[b].at[pl.ds(0, 3 * CH)],
            sem_m[b],
        )

    def _meta_wait(ci, b):
        off = (wid * NCHUNK + ci) * (3 * CH)
        pltpu.make_async_copy(
            meta_hbm.at[pl.ds(off, 3 * CH)],
            metab[b].at[pl.ds(0, 3 * CH)],
            sem_m[b],
        ).wait()

    def _data_load(ci, b):
        pltpu.async_copy(h_hbm.at[metab[b].at[pl.ds(0, CH)]], rows[b], sem_g)
        pltpu.async_copy(
            ef_hbm.at[pl.ds(ebase + ci * CH, CH), :], efb[b], sem_e
        )

    def _data_wait(ci, b):
        pltpu.make_async_copy(
            h_hbm.at[metab[b].at[pl.ds(0, CH)]], rows[b], sem_g
        ).wait()
        pltpu.make_async_copy(
            ef_hbm.at[pl.ds(ebase + ci * CH, CH), :], efb[b], sem_e
        ).wait()

    _meta_load(0, 0)
    _meta_load(1, 1)

    def _zero_row(r, carry):
        for c8 in range(D_OUT // LANES):
            zbuf[r, pl.ds(c8 * LANES, LANES)] = jnp.zeros((LANES,), jnp.float32)
        return carry

    lax.fori_loop(0, ZR, _zero_row, 0)
    r0 = pl.multiple_of(sid * ROWS_MAIN, ROWS_MAIN)

    @pl.when(sid < NS - 1)
    def _zero_main():
        for k in range(ROWS_MAIN // ZR):
            pltpu.sync_copy(zbuf, agg.at[pl.ds(r0 + k * ZR, ZR), :])

    @pl.when(sid == NS - 1)
    def _zero_last():
        base = (NS - 1) * ROWS_MAIN
        for k in range(ROWS_LAST // ZR):
            pltpu.sync_copy(zbuf, agg.at[pl.ds(base + k * ZR, ZR), :])

    plsc.subcore_barrier()

    _meta_wait(0, 0)
    _data_load(0, 0)

    def _compute(b):
        rv = rows[b]
        ev = efb[b]
        nb = metab[b]

        def _edge_grp(g, c2):
            e0 = g * 8
            nv = nb[pl.ds(2 * CH + e0, LANES)].astype(jnp.float32) * (2.0**-24)
            for j in range(8):
                e = e0 + j
                nsp = lax.broadcast(nv[j], (LANES,))
                for v in range(D_OUT // LANES):
                    x = rv[e, pl.ds(v * LANES, LANES)]
                    y = ev[e, pl.ds(v * LANES, LANES)]
                    rv[e, pl.ds(v * LANES, LANES)] = (
                        jnp.maximum(x + y, 0.0) * nsp
                    )
            return c2

        lax.fori_loop(0, CH // 8, _edge_grp, 0)

    def _body(ci, b, has1, has2):
        _data_wait(ci, b)
        if has1:
            _meta_wait(ci + 1, 1 - b)
            _data_load(ci + 1, 1 - b)
        _compute(b)
        pltpu.sync_copy(rows[b], agg.at[metab[b].at[pl.ds(CH, CH)]], add=True)
        if has2:
            _meta_load(ci + 2, b)

    _body(0, 0, True, True)

    def _group(j, carry):
        ci = 1 + 2 * j
        _body(ci, 1, True, True)
        _body(ci + 1, 0, True, True)
        return carry

    lax.fori_loop(0, (NCHUNK - 3) // 2, _group, 0)
    _body(NCHUNK - 2, 1, True, False)
    _body(NCHUNK - 1, 0, False, False)
    plsc.subcore_barrier()

    @pl.when(sid < NS - 1)
    def _pub_main():
        pltpu.sync_copy(
            agg.at[pl.ds(r0, ROWS_MAIN), :],
            out_hbm.at[cid, pl.ds(r0, ROWS_MAIN), :],
        )

    @pl.when(sid == NS - 1)
    def _pub_last():
        base = (NS - 1) * ROWS_MAIN
        pltpu.sync_copy(
            agg.at[pl.ds(base, ROWS_LAST), :],
            out_hbm.at[cid, pl.ds(base, ROWS_LAST), :],
        )


@functools.cache
def _make_sc_edge():
    return pl.kernel(
        _sc_edge_body,
        out_type=jax.ShapeDtypeStruct((NC, N, D_OUT), jnp.float32),
        mesh=plsc.VectorSubcoreMesh(
            core_axis_name="c", subcore_axis_name="s", num_cores=NC, num_subcores=NS
        ),
        scratch_types=[
            pltpu.VMEM((3 * CH + LANES,), jnp.int32),
            pltpu.VMEM((3 * CH + LANES,), jnp.int32),
            pltpu.VMEM((CH, D_OUT), jnp.float32),
            pltpu.VMEM((CH, D_OUT), jnp.float32),
            pltpu.VMEM((CH, D_OUT), jnp.float32),
            pltpu.VMEM((CH, D_OUT), jnp.float32),
            pltpu.VMEM((ZR, D_OUT), jnp.float32),
            pltpu.VMEM_SHARED((N, D_OUT), jnp.float32),
            pltpu.SemaphoreType.DMA,
            pltpu.SemaphoreType.DMA,
            pltpu.SemaphoreType.DMA,
            pltpu.SemaphoreType.DMA,
        ],
    )


@jax.jit
def kernel(node_feats, edge_feats, degs, norm, Wn, bn, We, be, res_w, edge_index):
    srcr = edge_index[0].reshape(NW, NCHUNK, 1, CH)
    dstr = edge_index[1].reshape(NW, NCHUNK, 1, CH)
    normi = (norm.reshape(NW, NCHUNK, 1, CH) * (2.0**24)).astype(jnp.int32)
    meta = jnp.concatenate([srcr, dstr, normi], axis=2).reshape(3 * E)
    bn2 = bn.reshape(1, D_OUT)
    be2 = be.reshape(1, D_OUT)

    _clamp = lambda i: (jnp.minimum(i, N // BN - 1), 0)
    h, ef = pl.pallas_call(
        _proj_body,
        grid=(E // BE,),
        in_specs=[
            pl.BlockSpec((BN, D_IN), _clamp),
            pl.BlockSpec((D_OUT, D_IN), lambda i: (0, 0)),
            pl.BlockSpec((1, D_OUT), lambda i: (0, 0)),
            pl.BlockSpec((BE, D_EDGE), lambda i: (i, 0)),
            pl.BlockSpec((D_OUT, D_EDGE), lambda i: (0, 0)),
            pl.BlockSpec((1, D_OUT), lambda i: (0, 0)),
        ],
        out_specs=[
            pl.BlockSpec((BN, D_OUT), _clamp),
            pl.BlockSpec((BE, D_OUT), lambda i: (i, 0)),
        ],
        out_shape=[
            jax.ShapeDtypeStruct((N, D_OUT), jnp.float32),
            jax.ShapeDtypeStruct((E, D_OUT), jnp.float32),
        ],
    )(node_feats, Wn, bn2, edge_feats, We, be2)

    partials = _make_sc_edge()(meta, ef, h)

    out = pl.pallas_call(
        _combine_body,
        grid=(N // BN,),
        in_specs=[
            pl.BlockSpec((NC, BN, D_OUT), lambda i: (0, i, 0)),
            pl.BlockSpec((BN, D_OUT), lambda i: (i, 0)),
            pl.BlockSpec((1, D_OUT), lambda i: (0, 0)),
            pl.BlockSpec((BN, 1), lambda i: (i, 0)),
        ],
        out_specs=pl.BlockSpec((BN, D_OUT), lambda i: (i, 0)),
        out_shape=jax.ShapeDtypeStruct((N, D_OUT), jnp.float32),
    )(partials, h, res_w, degs)

    return out


# speedup vs baseline: 1.4271x; 1.4271x over previous
import functools

import jax
import jax.numpy as jnp
from jax import lax
from jax.experimental import pallas as pl
from jax.experimental.pallas import tpu as pltpu
from jax.experimental.pallas import tpu_sc as plsc

N = 10000
E = 320000
D_IN = 128
D_EDGE = 16
D_OUT = 128

NC = 2
NS = 16
NW = NC * NS
LANES = 16

EW = E // NW
CH = 80
NCHUNK = EW // CH
ROWS_MAIN = 640
ROWS_LAST = N - (NS - 1) * ROWS_MAIN
ZR = 16
BN = 1000
BE = 16000


def _proj_body(nf_ref, wn_ref, bn_ref, ef_in_ref, we_ref, be_ref,
               h_ref, ef_ref):
    @pl.when(pl.program_id(0) < N // BN)
    def _():
        h_ref[...] = (
            lax.dot_general(
                nf_ref[...], wn_ref[...],
                (((1,), (1,)), ((), ())),
                preferred_element_type=jnp.float32,
            )
            + bn_ref[...]
        )

    ef_ref[...] = (
        lax.dot_general(
            ef_in_ref[...].astype(jnp.bfloat16),
            we_ref[...].astype(jnp.bfloat16),
            (((1,), (1,)), ((), ())),
            preferred_element_type=jnp.float32,
        )
        + be_ref[...]
    )


def _combine_body(p_ref, h_ref, resw_ref, degs_ref, o_ref):
    h = h_ref[...]
    res = jnp.maximum(h + resw_ref[...], 0.0) * (1.0 / degs_ref[...])
    o_ref[...] = p_ref[0] + p_ref[1] + res


def _sc_edge_body(src_hbm, dst_hbm, norm_hbm, ef_hbm, h_hbm, out_hbm,
                  srcb0, srcb1, normb0, normb1, dstb0, dstb1,
                  rows0, rows1, ef0, ef1, zbuf, agg,
                  sem_m0, sem_m1, sem_g, sem_e, sem_d0, sem_d1):
    cid = lax.axis_index("c")
    sid = lax.axis_index("s")
    wid = cid * NS + sid

    srcb = (srcb0, srcb1)
    normb = (normb0, normb1)
    dstb = (dstb0, dstb1)
    rows = (rows0, rows1)
    efb = (ef0, ef1)
    sem_m = (sem_m0, sem_m1)
    sem_d = (sem_d0, sem_d1)

    ebase = wid * EW

    def _meta_load(ci, b):
        off = ebase + ci * CH
        pltpu.async_copy(src_hbm.at[pl.ds(off, CH)], srcb[b], sem_m[b])
        pltpu.async_copy(
            norm_hbm.at[pl.ds(off, CH)], normb[b].at[pl.ds(0, CH)], sem_m[b]
        )

    def _meta_wait(ci, b):
        off = ebase + ci * CH
        pltpu.make_async_copy(
            src_hbm.at[pl.ds(off, CH)], srcb[b], sem_m[b]
        ).wait()
        pltpu.make_async_copy(
            norm_hbm.at[pl.ds(off, CH)], normb[b].at[pl.ds(0, CH)], sem_m[b]
        ).wait()

    def _dst_load(ci, b):
        off = ebase + ci * CH
        pltpu.async_copy(dst_hbm.at[pl.ds(off, CH)], dstb[b], sem_d[b])

    def _dst_wait(ci, b):
        off = ebase + ci * CH
        pltpu.make_async_copy(
            dst_hbm.at[pl.ds(off, CH)], dstb[b], sem_d[b]
        ).wait()

    def _data_load(ci, b):
        pltpu.async_copy(h_hbm.at[srcb[b]], rows[b], sem_g)
        pltpu.async_copy(
            ef_hbm.at[pl.ds(ebase + ci * CH, CH), :], efb[b], sem_e
        )

    def _data_wait(ci, b):
        pltpu.make_async_copy(h_hbm.at[srcb[b]], rows[b], sem_g).wait()
        pltpu.make_async_copy(
            ef_hbm.at[pl.ds(ebase + ci * CH, CH), :], efb[b], sem_e
        ).wait()

    _meta_load(0, 0)
    _meta_load(1, 1)
    _dst_load(0, 0)

    def _zero_row(r, carry):
        for c8 in range(D_OUT // LANES):
            zbuf[r, pl.ds(c8 * LANES, LANES)] = jnp.zeros((LANES,), jnp.float32)
        return carry

    lax.fori_loop(0, ZR, _zero_row, 0)
    r0 = pl.multiple_of(sid * ROWS_MAIN, ROWS_MAIN)

    @pl.when(sid < NS - 1)
    def _zero_main():
        for k in range(ROWS_MAIN // ZR):
            pltpu.sync_copy(zbuf, agg.at[pl.ds(r0 + k * ZR, ZR), :])

    @pl.when(sid == NS - 1)
    def _zero_last():
        base = (NS - 1) * ROWS_MAIN
        for k in range(ROWS_LAST // ZR):
            pltpu.sync_copy(zbuf, agg.at[pl.ds(base + k * ZR, ZR), :])

    plsc.subcore_barrier()

    _meta_wait(0, 0)
    _data_load(0, 0)

    def _compute(b):
        rv = rows[b]
        ev = efb[b]
        nb = normb[b]

        def _edge_grp(g, c2):
            e0 = g * 8
            nv = nb[pl.ds(e0, LANES)]
            for j in range(8):
                e = e0 + j
                nsp = lax.broadcast(nv[j], (LANES,))
                for v in range(D_OUT // LANES):
                    x = rv[e, pl.ds(v * LANES, LANES)]
                    y = ev[e, pl.ds(v * LANES, LANES)]
                    rv[e, pl.ds(v * LANES, LANES)] = (
                        jnp.maximum(x + y, 0.0) * nsp
                    )
            return c2

        lax.fori_loop(0, CH // 8, _edge_grp, 0)

    def _body(ci, b, has1, has2):
        _data_wait(ci, b)
        if has1:
            _meta_wait(ci + 1, 1 - b)
            _data_load(ci + 1, 1 - b)
            _dst_load(ci + 1, 1 - b)
        _compute(b)
        if has2:
            _meta_load(ci + 2, b)
        _dst_wait(ci, b)
        pltpu.sync_copy(rows[b], agg.at[dstb[b]], add=True)

    _body(0, 0, True, True)

    def _group(j, carry):
        ci = 1 + 2 * j
        _body(ci, 1, True, True)
        _body(ci + 1, 0, True, True)
        return carry

    lax.fori_loop(0, (NCHUNK - 3) // 2, _group, 0)
    _body(NCHUNK - 2, 1, True, False)
    _body(NCHUNK - 1, 0, False, False)
    plsc.subcore_barrier()

    @pl.when(sid < NS - 1)
    def _pub_main():
        pltpu.sync_copy(
            agg.at[pl.ds(r0, ROWS_MAIN), :],
            out_hbm.at[cid, pl.ds(r0, ROWS_MAIN), :],
        )

    @pl.when(sid == NS - 1)
    def _pub_last():
        base = (NS - 1) * ROWS_MAIN
        pltpu.sync_copy(
            agg.at[pl.ds(base, ROWS_LAST), :],
            out_hbm.at[cid, pl.ds(base, ROWS_LAST), :],
        )


@functools.cache
def _make_sc_edge():
    return pl.kernel(
        _sc_edge_body,
        out_type=jax.ShapeDtypeStruct((NC, N, D_OUT), jnp.float32),
        mesh=plsc.VectorSubcoreMesh(
            core_axis_name="c", subcore_axis_name="s", num_cores=NC, num_subcores=NS
        ),
        scratch_types=[
            pltpu.VMEM((CH,), jnp.int32),
            pltpu.VMEM((CH,), jnp.int32),
            pltpu.VMEM((CH + LANES,), jnp.float32),
            pltpu.VMEM((CH + LANES,), jnp.float32),
            pltpu.VMEM((CH,), jnp.int32),
            pltpu.VMEM((CH,), jnp.int32),
            pltpu.VMEM((CH, D_OUT), jnp.float32),
            pltpu.VMEM((CH, D_OUT), jnp.float32),
            pltpu.VMEM((CH, D_OUT), jnp.float32),
            pltpu.VMEM((CH, D_OUT), jnp.float32),
            pltpu.VMEM((ZR, D_OUT), jnp.float32),
            pltpu.VMEM_SHARED((N, D_OUT), jnp.float32),
            pltpu.SemaphoreType.DMA,
            pltpu.SemaphoreType.DMA,
            pltpu.SemaphoreType.DMA,
            pltpu.SemaphoreType.DMA,
            pltpu.SemaphoreType.DMA,
            pltpu.SemaphoreType.DMA,
        ],
    )


@jax.jit
def kernel(node_feats, edge_feats, degs, norm, Wn, bn, We, be, res_w, edge_index):
    src = edge_index[0]
    dst = edge_index[1]
    norm_flat = norm.reshape(E)
    bn2 = bn.reshape(1, D_OUT)
    be2 = be.reshape(1, D_OUT)

    _clamp = lambda i: (jnp.minimum(i, N // BN - 1), 0)
    h, ef = pl.pallas_call(
        _proj_body,
        grid=(E // BE,),
        in_specs=[
            pl.BlockSpec((BN, D_IN), _clamp),
            pl.BlockSpec((D_OUT, D_IN), lambda i: (0, 0)),
            pl.BlockSpec((1, D_OUT), lambda i: (0, 0)),
            pl.BlockSpec((BE, D_EDGE), lambda i: (i, 0)),
            pl.BlockSpec((D_OUT, D_EDGE), lambda i: (0, 0)),
            pl.BlockSpec((1, D_OUT), lambda i: (0, 0)),
        ],
        out_specs=[
            pl.BlockSpec((BN, D_OUT), _clamp),
            pl.BlockSpec((BE, D_OUT), lambda i: (i, 0)),
        ],
        out_shape=[
            jax.ShapeDtypeStruct((N, D_OUT), jnp.float32),
            jax.ShapeDtypeStruct((E, D_OUT), jnp.float32),
        ],
    )(node_feats, Wn, bn2, edge_feats, We, be2)

    partials = _make_sc_edge()(src, dst, norm_flat, ef, h)

    out = pl.pallas_call(
        _combine_body,
        grid=(N // BN,),
        in_specs=[
            pl.BlockSpec((NC, BN, D_OUT), lambda i: (0, i, 0)),
            pl.BlockSpec((BN, D_OUT), lambda i: (i, 0)),
            pl.BlockSpec((1, D_OUT), lambda i: (0, 0)),
            pl.BlockSpec((BN, 1), lambda i: (i, 0)),
        ],
        out_specs=pl.BlockSpec((BN, D_OUT), lambda i: (i, 0)),
        out_shape=jax.ShapeDtypeStruct((N, D_OUT), jnp.float32),
    )(partials, h, res_w, degs)

    return out


# speedup vs baseline: 1.4341x; 1.0050x over previous
import functools

import jax
import jax.numpy as jnp
from jax import lax
from jax.experimental import pallas as pl
from jax.experimental.pallas import tpu as pltpu
from jax.experimental.pallas import tpu_sc as plsc

N = 10000
E = 320000
D_IN = 128
D_EDGE = 16
D_OUT = 128

NC = 2
NS = 16
NW = NC * NS
LANES = 16

EW = E // NW
CH = 80
NCHUNK = EW // CH
ROWS_MAIN = 640
ROWS_LAST = N - (NS - 1) * ROWS_MAIN
ZR = 16
BN = 2000
BE = 16000


def _proj_body(nf_ref, wn_ref, bn_ref, ef_in_ref, we_ref, be_ref,
               h_ref, ef_ref):
    @pl.when(pl.program_id(0) < N // BN)
    def _():
        h_ref[...] = (
            lax.dot_general(
                nf_ref[...], wn_ref[...],
                (((1,), (1,)), ((), ())),
                preferred_element_type=jnp.float32,
            )
            + bn_ref[...]
        )

    ef_ref[...] = (
        lax.dot_general(
            ef_in_ref[...].astype(jnp.bfloat16),
            we_ref[...].astype(jnp.bfloat16),
            (((1,), (1,)), ((), ())),
            preferred_element_type=jnp.float32,
        )
        + be_ref[...]
    )


def _combine_body(p_ref, h_ref, resw_ref, degs_ref, o_ref):
    h = h_ref[...]
    res = jnp.maximum(h + resw_ref[...], 0.0) * (1.0 / degs_ref[...])
    o_ref[...] = p_ref[0] + p_ref[1] + res


def _sc_edge_body(src_hbm, dst_hbm, norm_hbm, ef_hbm, h_hbm, out_hbm,
                  srcb0, srcb1, normb0, normb1, dstb0, dstb1,
                  rows0, rows1, ef0, ef1, zbuf, agg,
                  sem_m0, sem_m1, sem_g, sem_e, sem_d0, sem_d1):
    cid = lax.axis_index("c")
    sid = lax.axis_index("s")
    wid = cid * NS + sid

    srcb = (srcb0, srcb1)
    normb = (normb0, normb1)
    dstb = (dstb0, dstb1)
    rows = (rows0, rows1)
    efb = (ef0, ef1)
    sem_m = (sem_m0, sem_m1)
    sem_d = (sem_d0, sem_d1)

    ebase = wid * EW

    def _meta_load(ci, b):
        off = ebase + ci * CH
        pltpu.async_copy(src_hbm.at[pl.ds(off, CH)], srcb[b], sem_m[b])
        pltpu.async_copy(
            norm_hbm.at[pl.ds(off, CH)], normb[b].at[pl.ds(0, CH)], sem_m[b]
        )

    def _meta_wait(ci, b):
        off = ebase + ci * CH
        pltpu.make_async_copy(
            src_hbm.at[pl.ds(off, CH)], srcb[b], sem_m[b]
        ).wait()
        pltpu.make_async_copy(
            norm_hbm.at[pl.ds(off, CH)], normb[b].at[pl.ds(0, CH)], sem_m[b]
        ).wait()

    def _dst_load(ci, b):
        off = ebase + ci * CH
        pltpu.async_copy(dst_hbm.at[pl.ds(off, CH)], dstb[b], sem_d[b])

    def _dst_wait(ci, b):
        off = ebase + ci * CH
        pltpu.make_async_copy(
            dst_hbm.at[pl.ds(off, CH)], dstb[b], sem_d[b]
        ).wait()

    def _data_load(ci, b):
        pltpu.async_copy(h_hbm.at[srcb[b]], rows[b], sem_g)
        pltpu.async_copy(
            ef_hbm.at[pl.ds(ebase + ci * CH, CH), :], efb[b], sem_e
        )

    def _data_wait(ci, b):
        pltpu.make_async_copy(h_hbm.at[srcb[b]], rows[b], sem_g).wait()
        pltpu.make_async_copy(
            ef_hbm.at[pl.ds(ebase + ci * CH, CH), :], efb[b], sem_e
        ).wait()

    _meta_load(0, 0)
    _meta_load(1, 1)
    _dst_load(0, 0)

    def _zero_row(r, carry):
        for c8 in range(D_OUT // LANES):
            zbuf[r, pl.ds(c8 * LANES, LANES)] = jnp.zeros((LANES,), jnp.float32)
        return carry

    lax.fori_loop(0, ZR, _zero_row, 0)
    r0 = pl.multiple_of(sid * ROWS_MAIN, ROWS_MAIN)

    @pl.when(sid < NS - 1)
    def _zero_main():
        for k in range(ROWS_MAIN // ZR):
            pltpu.sync_copy(zbuf, agg.at[pl.ds(r0 + k * ZR, ZR), :])

    @pl.when(sid == NS - 1)
    def _zero_last():
        base = (NS - 1) * ROWS_MAIN
        for k in range(ROWS_LAST // ZR):
            pltpu.sync_copy(zbuf, agg.at[pl.ds(base + k * ZR, ZR), :])

    plsc.subcore_barrier()

    _meta_wait(0, 0)
    _data_load(0, 0)

    def _compute(b):
        rv = rows[b]
        ev = efb[b]
        nb = normb[b]

        def _edge_grp(g, c2):
            e0 = g * 8
            nv = nb[pl.ds(e0, LANES)]
            for j in range(8):
                e = e0 + j
                nsp = lax.broadcast(nv[j], (LANES,))
                for v in range(D_OUT // LANES):
                    x = rv[e, pl.ds(v * LANES, LANES)]
                    y = ev[e, pl.ds(v * LANES, LANES)]
                    rv[e, pl.ds(v * LANES, LANES)] = (
                        jnp.maximum(x + y, 0.0) * nsp
                    )
            return c2

        lax.fori_loop(0, CH // 8, _edge_grp, 0)

    def _body(ci, b, has1, has2):
        _data_wait(ci, b)
        if has1:
            _meta_wait(ci + 1, 1 - b)
            _data_load(ci + 1, 1 - b)
            _dst_load(ci + 1, 1 - b)
        _compute(b)
        if has2:
            _meta_load(ci + 2, b)
        _dst_wait(ci, b)
        pltpu.sync_copy(rows[b], agg.at[dstb[b]], add=True)

    _body(0, 0, True, True)

    def _group(j, carry):
        ci = 1 + 2 * j
        _body(ci, 1, True, True)
        _body(ci + 1, 0, True, True)
        return carry

    lax.fori_loop(0, (NCHUNK - 3) // 2, _group, 0)
    _body(NCHUNK - 2, 1, True, False)
    _body(NCHUNK - 1, 0, False, False)
    plsc.subcore_barrier()

    @pl.when(sid < NS - 1)
    def _pub_main():
        pltpu.sync_copy(
            agg.at[pl.ds(r0, ROWS_MAIN), :],
            out_hbm.at[cid, pl.ds(r0, ROWS_MAIN), :],
        )

    @pl.when(sid == NS - 1)
    def _pub_last():
        base = (NS - 1) * ROWS_MAIN
        pltpu.sync_copy(
            agg.at[pl.ds(base, ROWS_LAST), :],
            out_hbm.at[cid, pl.ds(base, ROWS_LAST), :],
        )


@functools.cache
def _make_sc_edge():
    return pl.kernel(
        _sc_edge_body,
        out_type=jax.ShapeDtypeStruct((NC, N, D_OUT), jnp.float32),
        mesh=plsc.VectorSubcoreMesh(
            core_axis_name="c", subcore_axis_name="s", num_cores=NC, num_subcores=NS
        ),
        scratch_types=[
            pltpu.VMEM((CH,), jnp.int32),
            pltpu.VMEM((CH,), jnp.int32),
            pltpu.VMEM((CH + LANES,), jnp.float32),
            pltpu.VMEM((CH + LANES,), jnp.float32),
            pltpu.VMEM((CH,), jnp.int32),
            pltpu.VMEM((CH,), jnp.int32),
            pltpu.VMEM((CH, D_OUT), jnp.float32),
            pltpu.VMEM((CH, D_OUT), jnp.float32),
            pltpu.VMEM((CH, D_OUT), jnp.float32),
            pltpu.VMEM((CH, D_OUT), jnp.float32),
            pltpu.VMEM((ZR, D_OUT), jnp.float32),
            pltpu.VMEM_SHARED((N, D_OUT), jnp.float32),
            pltpu.SemaphoreType.DMA,
            pltpu.SemaphoreType.DMA,
            pltpu.SemaphoreType.DMA,
            pltpu.SemaphoreType.DMA,
            pltpu.SemaphoreType.DMA,
            pltpu.SemaphoreType.DMA,
        ],
    )


@jax.jit
def kernel(node_feats, edge_feats, degs, norm, Wn, bn, We, be, res_w, edge_index):
    src = edge_index[0]
    dst = edge_index[1]
    norm_flat = norm.reshape(E)
    bn2 = bn.reshape(1, D_OUT)
    be2 = be.reshape(1, D_OUT)

    _clamp = lambda i: (jnp.minimum(i, N // BN - 1), 0)
    h, ef = pl.pallas_call(
        _proj_body,
        grid=(E // BE,),
        in_specs=[
            pl.BlockSpec((BN, D_IN), _clamp),
            pl.BlockSpec((D_OUT, D_IN), lambda i: (0, 0)),
            pl.BlockSpec((1, D_OUT), lambda i: (0, 0)),
            pl.BlockSpec((BE, D_EDGE), lambda i: (i, 0)),
            pl.BlockSpec((D_OUT, D_EDGE), lambda i: (0, 0)),
            pl.BlockSpec((1, D_OUT), lambda i: (0, 0)),
        ],
        out_specs=[
            pl.BlockSpec((BN, D_OUT), _clamp),
            pl.BlockSpec((BE, D_OUT), lambda i: (i, 0)),
        ],
        out_shape=[
            jax.ShapeDtypeStruct((N, D_OUT), jnp.float32),
            jax.ShapeDtypeStruct((E, D_OUT), jnp.float32),
        ],
    )(node_feats, Wn, bn2, edge_feats, We, be2)

    partials = _make_sc_edge()(src, dst, norm_flat, ef, h)

    out = pl.pallas_call(
        _combine_body,
        grid=(N // BN,),
        in_specs=[
            pl.BlockSpec((NC, BN, D_OUT), lambda i: (0, i, 0)),
            pl.BlockSpec((BN, D_OUT), lambda i: (i, 0)),
            pl.BlockSpec((1, D_OUT), lambda i: (0, 0)),
            pl.BlockSpec((BN, 1), lambda i: (i, 0)),
        ],
        out_specs=pl.BlockSpec((BN, D_OUT), lambda i: (i, 0)),
        out_shape=jax.ShapeDtypeStruct((N, D_OUT), jnp.float32),
    )(partials, h, res_w, degs)

    return out
